# baseline (speedup 1.0000x reference)
import functools

import jax
import jax.numpy as jnp
from jax.experimental import pallas as pl
from jax.experimental.pallas import tpu as pltpu

_B = 8
_P = 256
_D = 384
_M = 20000
_G = 128
_NB = 10
_MBLK = _M // _NB
_ALPHA = 0.7


def _body(q_ref, mb_ref, g_ref, mbg_ref, out_ref, qn_s, a2_s, m_s, *, nb_total):
    nb = pl.program_id(0)
    b = pl.program_id(1)

    @pl.when(nb == 0)
    def _init():
        q = q_ref[...]
        nrm = jnp.sqrt(jnp.sum(q * q, axis=1, keepdims=True))
        qn = q / (nrm + 1e-12)
        qn_s[b] = qn
        a2_s[b] = jnp.sum(qn * qn, axis=1, keepdims=True)
        m_s[b] = jnp.full((_P, 1), jnp.inf, jnp.float32)

    mb = mb_ref[...]
    ones = jnp.ones((1, _D), jnp.float32)
    b2 = jax.lax.dot_general(ones, mb * mb, (((1,), (1,)), ((), ())),
                             preferred_element_type=jnp.float32)
    qb = qn_s[b].astype(jnp.bfloat16)
    t = jax.lax.dot_general(qb, mb.astype(jnp.bfloat16),
                            (((1,), (1,)), ((), ())),
                            preferred_element_type=jnp.float32)
    t2 = b2 - 2.0 * t
    m_s[b] = jnp.minimum(m_s[b], jnp.min(t2, axis=1, keepdims=True))

    @pl.when(nb == nb_total - 1)
    def _fini():
        d2max = jnp.max(m_s[b] + a2_s[b])
        local = jnp.sqrt(jnp.maximum(d2max, 0.0))
        g = g_ref[pl.ds(b, 1), :]
        gn = g / (jnp.sqrt(jnp.sum(g * g)) + 1e-12)
        gsq = jnp.sum(gn * gn)
        mbg = mbg_ref[...]
        bg2 = jax.lax.dot_general(ones, mbg * mbg, (((1,), (1,)), ((), ())),
                                  preferred_element_type=jnp.float32)
        tg = jax.lax.dot_general(gn, mbg, (((1,), (1,)), ((), ())),
                                 preferred_element_type=jnp.float32)
        gmin = jnp.min(bg2 - 2.0 * tg)
        gdist = jnp.sqrt(jnp.maximum(gsq + gmin, 0.0))
        out_ref[b] = _ALPHA * local + (1.0 - _ALPHA) * gdist


def kernel(patches, global_feat, mb_local, mb_global):
    q = patches.reshape(_B * _P, _D)
    return pl.pallas_call(
        functools.partial(_body, nb_total=_NB),
        grid=(_NB, _B),
        in_specs=[
            pl.BlockSpec((_P, _D), lambda nb, b: (b, 0)),
            pl.BlockSpec((_MBLK, _D), lambda nb, b: (nb, 0)),
            pl.BlockSpec((_B, _D), lambda nb, b: (0, 0)),
            pl.BlockSpec((_G, _D), lambda nb, b: (0, 0)),
        ],
        out_specs=pl.BlockSpec(memory_space=pltpu.SMEM),
        out_shape=jax.ShapeDtypeStruct((_B,), jnp.float32),
        scratch_shapes=[
            pltpu.VMEM((_B, _P, _D), jnp.float32),
            pltpu.VMEM((_B, _P, 1), jnp.float32),
            pltpu.VMEM((_B, _P, 1), jnp.float32),
        ],
        compiler_params=pltpu.CompilerParams(
            dimension_semantics=("arbitrary", "arbitrary")),
    )(q, mb_local, global_feat, mb_global)


# speedup vs baseline: 2.1758x; 2.1758x over previous
import functools

import jax
import jax.numpy as jnp
from jax.experimental import pallas as pl
from jax.experimental.pallas import tpu as pltpu

_B = 8
_P = 256
_D = 384
_M = 20000
_G = 128
_NB = 10
_MBLK = _M // _NB
_ALPHA = 0.7


def _body(q_ref, mb_ref, g_ref, mbg_ref, out_ref, qn_s, a2_s, m_s, *, nb_total):
    nb = pl.program_id(0)

    @pl.when(nb == 0)
    def _init():
        q = q_ref[...]
        nrm = jnp.sqrt(jnp.sum(q * q, axis=1, keepdims=True))
        qn = q / (nrm + 1e-12)
        qn_s[...] = qn.astype(jnp.bfloat16)
        a2_s[...] = jnp.sum(qn * qn, axis=1, keepdims=True)
        m_s[...] = jnp.full((_B * _P, 1), jnp.inf, jnp.float32)

    mb = mb_ref[...]
    ones = jnp.ones((1, _D), jnp.float32)
    b2h = jax.lax.dot_general(ones * 0.5, mb * mb, (((1,), (1,)), ((), ())),
                              preferred_element_type=jnp.float32)
    t = jax.lax.dot_general(qn_s[...], mb.astype(jnp.bfloat16),
                            (((1,), (1,)), ((), ())),
                            preferred_element_type=jnp.float32)
    s = b2h - t
    m_s[...] = jnp.minimum(m_s[...], jnp.min(s, axis=1, keepdims=True))

    @pl.when(nb == nb_total - 1)
    def _fini():
        g = g_ref[...]
        gn = g / (jnp.sqrt(jnp.sum(g * g, axis=1, keepdims=True)) + 1e-12)
        gsq = jnp.sum(gn * gn, axis=1, keepdims=True)
        mbg = mbg_ref[...]
        bg2 = jax.lax.dot_general(ones, mbg * mbg, (((1,), (1,)), ((), ())),
                                  preferred_element_type=jnp.float32)
        tg = jax.lax.dot_general(gn, mbg, (((1,), (1,)), ((), ())),
                                 preferred_element_type=jnp.float32)
        gmin = jnp.min(bg2 - 2.0 * tg, axis=1, keepdims=True) + gsq
        d2 = a2_s[...] + 2.0 * m_s[...]
        for b in range(_B):
            d2max = jnp.max(d2[b * _P:(b + 1) * _P, :])
            local = jnp.sqrt(jnp.maximum(d2max, 0.0))
            gdist = jnp.sqrt(jnp.maximum(gmin[b, 0], 0.0))
            out_ref[b] = _ALPHA * local + (1.0 - _ALPHA) * gdist


def kernel(patches, global_feat, mb_local, mb_global):
    q = patches.reshape(_B * _P, _D)
    return pl.pallas_call(
        functools.partial(_body, nb_total=_NB),
        grid=(_NB,),
        in_specs=[
            pl.BlockSpec((_B * _P, _D), lambda nb: (0, 0)),
            pl.BlockSpec((_MBLK, _D), lambda nb: (nb, 0)),
            pl.BlockSpec((_B, _D), lambda nb: (0, 0)),
            pl.BlockSpec((_G, _D), lambda nb: (0, 0)),
        ],
        out_specs=pl.BlockSpec(memory_space=pltpu.SMEM),
        out_shape=jax.ShapeDtypeStruct((_B,), jnp.float32),
        scratch_shapes=[
            pltpu.VMEM((_B * _P, _D), jnp.bfloat16),
            pltpu.VMEM((_B * _P, 1), jnp.float32),
            pltpu.VMEM((_B * _P, 1), jnp.float32),
        ],
        compiler_params=pltpu.CompilerParams(
            dimension_semantics=("arbitrary",)),
    )(q, mb_local, global_feat, mb_global)
